# baseline (speedup 1.0000x reference)
import functools

import jax
import jax.numpy as jnp
from jax import lax
from jax.experimental import pallas as pl
from jax.experimental.pallas import tpu as pltpu
from jax.experimental.pallas import tpu_sc as plsc

N_NODES = 10000
D = 128
E_EDGES = 320000

NC, NS = 2, 16
NW = NC * NS
K = 128
CB = 8
C = 80
B = C // CB
EP = NW * C * K
EPW = EP // NW
NP = 10240
ROWS_PER_TILE = NP // NS
CNT_W = 128

_MESH = plsc.VectorSubcoreMesh(core_axis_name="c", subcore_axis_name="s")


@functools.partial(
    pl.kernel,
    out_type=jax.ShapeDtypeStruct((NC, NP, D), jnp.float32),
    mesh=_MESH,
    scratch_types=[
        pltpu.VMEM((C, K), jnp.int32),
        pltpu.VMEM((CB, K), jnp.int32),
        pltpu.VMEM((CB, K), jnp.int32),
        pltpu.VMEM((K, D), jnp.float32),
        pltpu.VMEM((K, D), jnp.float32),
        pltpu.SemaphoreType.DMA,
        pltpu.SemaphoreType.DMA,
        pltpu.SemaphoreType.DMA,
        pltpu.SemaphoreType.DMA,
        pltpu.VMEM_SHARED((NP, D), jnp.float32),
    ],
)
def _sc_agg(x_hbm, src_hbm, dst_hbm, zd_hbm, agg_hbm,
            src_v, dib0, dib1, rb0, rb1, ds0, ds1, gs0, gs1, acc):
  c = lax.axis_index("c")
  s = lax.axis_index("s")
  wid = c * NS + s
  r0 = s * ROWS_PER_TILE
  rbs, gss, dibs, dss = (rb0, rb1), (gs0, gs1), (dib0, dib1), (ds0, ds1)

  pltpu.sync_copy(zd_hbm.at[pl.ds(r0, ROWS_PER_TILE)],
                  acc.at[pl.ds(r0, ROWS_PER_TILE)])
  pltpu.sync_copy(src_hbm.at[wid], src_v)
  pltpu.async_copy(dst_hbm.at[wid, pl.ds(0, CB)], dib0, ds0)
  pltpu.async_copy(dst_hbm.at[wid, pl.ds(CB, CB)], dib1, ds1)
  pltpu.async_copy(x_hbm.at[src_v.at[0]], rb0, gs0)
  plsc.subcore_barrier()

  def do_block(b, p, prefetch):
    pltpu.make_async_copy(dst_hbm.at[wid, pl.ds(0, CB)],
                          dibs[p], dss[p]).wait()
    for jj in range(CB):
      g = b * CB + jj
      nxt = jnp.minimum(g + 1, C - 1)
      pltpu.async_copy(x_hbm.at[src_v.at[nxt]],
                       rbs[(jj + 1) % 2], gss[(jj + 1) % 2])
      pltpu.make_async_copy(x_hbm.at[src_v.at[0]],
                            rbs[jj % 2], gss[jj % 2]).wait()
      pltpu.sync_copy(rbs[jj % 2], acc.at[dibs[p].at[jj]], add=True)
    if prefetch:
      pltpu.async_copy(dst_hbm.at[wid, pl.ds((b + 2) * CB, CB)],
                       dibs[p], dss[p])

  def body(m, carry):
    do_block(m * 2, 0, True)
    do_block(m * 2 + 1, 1, True)
    return carry

  lax.fori_loop(0, B // 2 - 1, body, 0)
  do_block(B - 2, 0, False)
  do_block(B - 1, 1, False)
  pltpu.make_async_copy(x_hbm.at[src_v.at[0]], rb0, gs0).wait()

  plsc.subcore_barrier()
  pltpu.sync_copy(acc.at[pl.ds(r0, ROWS_PER_TILE)],
                  agg_hbm.at[c, pl.ds(r0, ROWS_PER_TILE)])


@functools.partial(
    pl.kernel,
    out_type=jax.ShapeDtypeStruct((NC, NP, D), jnp.float32),
    mesh=_MESH,
    scratch_types=[
        pltpu.VMEM((C, K), jnp.int32),
        pltpu.VMEM((C, K), jnp.int32),
        pltpu.VMEM((K, D), jnp.float32),
        pltpu.SemaphoreType.DMA,
        pltpu.VMEM_SHARED((NP, D), jnp.float32),
    ],
)
def _sc_agg_s(x_hbm, src_hbm, dst_hbm, zd_hbm, agg_hbm,
              src_v, dst_v, rb, gs, acc):
  c = lax.axis_index("c")
  s = lax.axis_index("s")
  wid = c * NS + s
  r0 = s * ROWS_PER_TILE

  pltpu.sync_copy(zd_hbm.at[pl.ds(r0, ROWS_PER_TILE)],
                  acc.at[pl.ds(r0, ROWS_PER_TILE)])
  pltpu.sync_copy(src_hbm.at[wid], src_v)
  pltpu.sync_copy(dst_hbm.at[wid], dst_v)
  plsc.subcore_barrier()

  def chunk(j, carry):
    pltpu.async_copy(x_hbm.at[src_v.at[j]], rb, gs).wait()
    pltpu.sync_copy(rb, acc.at[dst_v.at[j]], add=True)
    return carry

  lax.fori_loop(0, C, chunk, 0)
  plsc.subcore_barrier()
  pltpu.sync_copy(acc.at[pl.ds(r0, ROWS_PER_TILE)],
                  agg_hbm.at[c, pl.ds(r0, ROWS_PER_TILE)])


NR = NP // 128


@functools.partial(
    pl.kernel,
    out_type=jax.ShapeDtypeStruct((NC, NR, 128), jnp.float32),
    mesh=_MESH,
    compiler_params=pltpu.CompilerParams(needs_layout_passes=False),
    scratch_types=[
        pltpu.VMEM((C, K), jnp.int32),
        pltpu.VMEM((NR, 128), jnp.float32),
        pltpu.VMEM((1, NR), jnp.int32),
        pltpu.VMEM_SHARED((NR, 128), jnp.float32),
    ],
)
def _sc_cnt(dst_hbm, zd_hbm, iota_hbm, cnt_hbm, dst_v, cl_v, io_v, cacc):
  c = lax.axis_index("c")
  s = lax.axis_index("s")
  wid = c * NS + s

  pltpu.sync_copy(dst_hbm.at[wid], dst_v)
  pltpu.sync_copy(zd_hbm.at[pl.ds(0, NR)], cl_v)
  pltpu.sync_copy(iota_hbm, io_v)

  @pl.when(s == 0)
  def _():
    pltpu.sync_copy(zd_hbm.at[pl.ds(0, NR)], cacc)

  plsc.subcore_barrier()

  ones16 = jnp.ones((16,), jnp.float32)

  def chunk(j, carry):
    def sub(b, cy):
      idx = dst_v[j, pl.ds(b * 16, 16)]
      plsc.addupdate_scatter(
          cl_v,
          [lax.shift_right_logical(idx, 7), lax.bitwise_and(idx, 127)],
          ones16)
      return cy

    return lax.fori_loop(0, K // 16, sub, carry)

  lax.fori_loop(0, C, chunk, 0)
  pltpu.sync_copy(cl_v, cacc.at[io_v.at[0]], add=True)
  plsc.subcore_barrier()

  @pl.when(s == 0)
  def _():
    pltpu.sync_copy(cacc, cnt_hbm.at[c])


def _tc_layer(x, aggp, cntp, W_l, b_l, W_r, relu: bool):
  R = 1000
  grid = (N_NODES // R,)

  def body(x_ref, agg_ref, cnt_ref, wl_ref, wr_ref, b_ref, o_ref):
    agg = agg_ref[0] + agg_ref[1]
    mean = agg / jnp.maximum(cnt_ref[...], 1.0)
    dn = (((1,), (1,)), ((), ()))
    out = (lax.dot_general(mean, wl_ref[...], dn,
                           preferred_element_type=jnp.float32)
           + lax.dot_general(x_ref[...], wr_ref[...], dn,
                             preferred_element_type=jnp.float32)
           + b_ref[...])
    if relu:
      out = jnp.maximum(out, 0.0)
    o_ref[...] = out

  return pl.pallas_call(
      body,
      grid=grid,
      in_specs=[
          pl.BlockSpec((R, D), lambda i: (i, 0)),
          pl.BlockSpec((NC, R, D), lambda i: (0, i, 0)),
          pl.BlockSpec((R, 1), lambda i: (i, 0)),
          pl.BlockSpec((D, D), lambda i: (0, 0)),
          pl.BlockSpec((D, D), lambda i: (0, 0)),
          pl.BlockSpec((1, D), lambda i: (0, 0)),
      ],
      out_specs=pl.BlockSpec((R, D), lambda i: (i, 0)),
      out_shape=jax.ShapeDtypeStruct((N_NODES, D), jnp.float32),
  )(x, aggp, cntp, W_l, W_r, b_l.reshape(1, D))


def kernel(x, edge_index, W1_l, b1_l, W1_r, W2_l, b2_l, W2_r):
  pad = EP - E_EDGES
  srcp = jnp.concatenate([edge_index[0], jnp.zeros((pad,), jnp.int32)])
  pad_dst = N_NODES + (jnp.arange(pad, dtype=jnp.int32) % (NP - N_NODES))
  dstp = jnp.concatenate([edge_index[1], pad_dst])
  src3 = srcp.reshape(NW, C, K)
  dst3 = dstp.reshape(NW, C, K)
  zd = jnp.zeros((NP, D), jnp.float32)
  iota = jnp.arange(NR, dtype=jnp.int32).reshape(1, NR)

  cntp = _sc_cnt(dst3, zd, iota)
  cnt = (cntp[0] + cntp[1]).reshape(NP, 1)
  agg1p = _sc_agg_s(x, src3, dst3, zd)
  h = _tc_layer(x, agg1p, cnt, W1_l, b1_l, W1_r, relu=True)
  agg2p = _sc_agg_s(h, src3, dst3, zd)
  return _tc_layer(h, agg2p, cnt, W2_l, b2_l, W2_r, relu=False)


# speedup vs baseline: 2.2551x; 2.2551x over previous
import functools

import jax
import jax.numpy as jnp
from jax import lax
from jax.experimental import pallas as pl
from jax.experimental.pallas import tpu as pltpu
from jax.experimental.pallas import tpu_sc as plsc

N_NODES = 10000
D = 128
E_EDGES = 320000

NC, NS = 2, 16
NW = NC * NS
K = 80
C = 125
NP = 10240
ROWS_PER_TILE = NP // NS
NR = NP // 128

_MESH = plsc.VectorSubcoreMesh(core_axis_name="c", subcore_axis_name="s")


@functools.partial(
    pl.kernel,
    out_type=jax.ShapeDtypeStruct((NC, NP, D), jnp.float32),
    mesh=_MESH,
    scratch_types=[
        pltpu.VMEM((C, K), jnp.int32),
        pltpu.VMEM((C, K), jnp.int32),
        pltpu.VMEM((K, D), jnp.float32),
        pltpu.SemaphoreType.DMA,
        pltpu.VMEM_SHARED((NP, D), jnp.float32),
    ],
)
def _sc_agg(x_hbm, src_hbm, dst_hbm, zd_hbm, agg_hbm,
            src_v, dst_v, rb, gs, acc):
  c = lax.axis_index("c")
  s = lax.axis_index("s")
  wid = c * NS + s
  r0 = s * ROWS_PER_TILE

  pltpu.sync_copy(zd_hbm.at[pl.ds(r0, ROWS_PER_TILE)],
                  acc.at[pl.ds(r0, ROWS_PER_TILE)])
  pltpu.sync_copy(src_hbm.at[wid], src_v)
  pltpu.sync_copy(dst_hbm.at[wid], dst_v)
  plsc.subcore_barrier()

  def chunk(j, carry):
    pltpu.async_copy(x_hbm.at[src_v.at[j]], rb, gs).wait()
    pltpu.sync_copy(rb, acc.at[dst_v.at[j]], add=True)
    return carry

  lax.fori_loop(0, C, chunk, 0)
  plsc.subcore_barrier()
  pltpu.sync_copy(acc.at[pl.ds(r0, ROWS_PER_TILE)],
                  agg_hbm.at[c, pl.ds(r0, ROWS_PER_TILE)])


@functools.partial(
    pl.kernel,
    out_type=jax.ShapeDtypeStruct((NC, NR, 128), jnp.float32),
    mesh=_MESH,
    compiler_params=pltpu.CompilerParams(needs_layout_passes=False),
    scratch_types=[
        pltpu.VMEM((C, K), jnp.int32),
        pltpu.VMEM((NR, 128), jnp.float32),
        pltpu.VMEM((1, NR), jnp.int32),
        pltpu.VMEM_SHARED((NR, 128), jnp.float32),
    ],
)
def _sc_cnt(dst_hbm, zd_hbm, iota_hbm, cnt_hbm, dst_v, cl_v, io_v, cacc):
  c = lax.axis_index("c")
  s = lax.axis_index("s")
  wid = c * NS + s

  pltpu.sync_copy(dst_hbm.at[wid], dst_v)
  pltpu.sync_copy(zd_hbm.at[pl.ds(0, NR)], cl_v)
  pltpu.sync_copy(iota_hbm, io_v)

  @pl.when(s == 0)
  def _():
    pltpu.sync_copy(zd_hbm.at[pl.ds(0, NR)], cacc)

  plsc.subcore_barrier()

  ones16 = jnp.ones((16,), jnp.float32)

  def chunk(j, carry):
    def sub(b, cy):
      idx = dst_v[j, pl.ds(b * 16, 16)]
      plsc.addupdate_scatter(
          cl_v,
          [lax.shift_right_logical(idx, 7), lax.bitwise_and(idx, 127)],
          ones16)
      return cy

    return lax.fori_loop(0, K // 16, sub, carry)

  lax.fori_loop(0, C, chunk, 0)
  pltpu.sync_copy(cl_v, cacc.at[io_v.at[0]], add=True)
  plsc.subcore_barrier()

  @pl.when(s == 0)
  def _():
    pltpu.sync_copy(cacc, cnt_hbm.at[c])


def _tc_layer(x, aggp, cnt, W_l, b_l, W_r, relu: bool):
  R = 1000
  grid = (N_NODES // R,)

  def body(x_ref, agg_ref, cnt_ref, wl_ref, wr_ref, b_ref, o_ref):
    agg = agg_ref[0] + agg_ref[1]
    mean = agg / jnp.maximum(cnt_ref[...], 1.0)
    dn = (((1,), (1,)), ((), ()))
    out = (lax.dot_general(mean, wl_ref[...], dn,
                           preferred_element_type=jnp.float32)
           + lax.dot_general(x_ref[...], wr_ref[...], dn,
                             preferred_element_type=jnp.float32)
           + b_ref[...])
    if relu:
      out = jnp.maximum(out, 0.0)
    o_ref[...] = out

  return pl.pallas_call(
      body,
      grid=grid,
      in_specs=[
          pl.BlockSpec((R, D), lambda i: (i, 0)),
          pl.BlockSpec((NC, R, D), lambda i: (0, i, 0)),
          pl.BlockSpec((R, 1), lambda i: (i, 0)),
          pl.BlockSpec((D, D), lambda i: (0, 0)),
          pl.BlockSpec((D, D), lambda i: (0, 0)),
          pl.BlockSpec((1, D), lambda i: (0, 0)),
      ],
      out_specs=pl.BlockSpec((R, D), lambda i: (i, 0)),
      out_shape=jax.ShapeDtypeStruct((N_NODES, D), jnp.float32),
  )(x, aggp, cnt, W_l, W_r, b_l.reshape(1, D))


def kernel(x, edge_index, W1_l, b1_l, W1_r, W2_l, b2_l, W2_r):
  src3 = edge_index[0].reshape(NW, C, K)
  dst3 = edge_index[1].reshape(NW, C, K)
  zd = jnp.zeros((NP, D), jnp.float32)
  iota = jnp.arange(NR, dtype=jnp.int32).reshape(1, NR)

  cntp = _sc_cnt(dst3, zd, iota)
  cnt = (cntp[0] + cntp[1]).reshape(NP, 1)
  agg1p = _sc_agg(x, src3, dst3, zd)
  h = _tc_layer(x, agg1p, cnt, W1_l, b1_l, W1_r, relu=True)
  agg2p = _sc_agg(h, src3, dst3, zd)
  return _tc_layer(h, agg2p, cnt, W2_l, b2_l, W2_r, relu=False)
